# baseline (speedup 1.0000x reference)
import jax
import jax.numpy as jnp
from jax import lax
from jax.experimental import pallas as pl
from jax.experimental.pallas import tpu as pltpu

N_ATOMS = 32768
N_GHOST = 8192


def _loss_kernel(w_ref, g_ref, out_ref):
    hi = g_ref[...].reshape(1, N_GHOST) >> 7
    lo = g_ref[...].reshape(1, N_GHOST) & 127
    h_iota = lax.broadcasted_iota(jnp.int32, (256, N_GHOST), 0)
    hit = (h_iota == jnp.broadcast_to(hi, (256, N_GHOST))).astype(jnp.bfloat16)
    l_iota = lax.broadcasted_iota(jnp.int32, (128, N_GHOST), 0)
    lot = (l_iota == jnp.broadcast_to(lo, (128, N_GHOST))).astype(jnp.bfloat16)
    counts = jax.lax.dot_general(
        hit, lot, (((1,), (1,)), ((), ())),
        preferred_element_type=jnp.float32,
    )
    keep = (counts == 0.0).astype(jnp.float32)
    out_ref[0, 0] = jnp.sum(keep * w_ref[...]) * (1.0 / N_ATOMS)


@jax.jit
def kernel(pred_frac_eps_x, target_frac_eps_x, ghost_atom_indices):
    d = jnp.abs(pred_frac_eps_x - target_frac_eps_x)
    r = d - jnp.floor(d)
    w = jnp.minimum(r, 1.0 - r)
    s_row = jnp.sum(w * w, axis=1).reshape(256, 128)
    gidx = ghost_atom_indices.astype(jnp.int32)

    out = pl.pallas_call(
        _loss_kernel,
        out_shape=jax.ShapeDtypeStruct((1, 1), jnp.float32),
        out_specs=pl.BlockSpec(memory_space=pltpu.SMEM),
    )(s_row, gidx)
    return out.reshape(())


# speedup vs baseline: 1.0027x; 1.0027x over previous
import jax
import jax.numpy as jnp
from jax import lax
from jax.experimental import pallas as pl
from jax.experimental.pallas import tpu as pltpu

N_ATOMS = 32768
N_GHOST = 8192


def _loss_kernel(w_ref, g_ref, out_ref):
    hi = g_ref[...].reshape(1, N_GHOST) >> 7
    lo = g_ref[...].reshape(1, N_GHOST) & 127
    h_iota = lax.broadcasted_iota(jnp.int32, (256, N_GHOST), 0)
    hit = (h_iota == jnp.broadcast_to(hi, (256, N_GHOST))).astype(jnp.float32)
    l_iota = lax.broadcasted_iota(jnp.int32, (128, N_GHOST), 0)
    lot = (l_iota == jnp.broadcast_to(lo, (128, N_GHOST))).astype(jnp.float32)
    counts = jax.lax.dot_general(
        hit, lot, (((1,), (1,)), ((), ())),
        preferred_element_type=jnp.float32,
    )
    keep = (counts == 0.0).astype(jnp.float32)
    out_ref[0, 0] = jnp.sum(keep * w_ref[...]) * (1.0 / N_ATOMS)


@jax.jit
def kernel(pred_frac_eps_x, target_frac_eps_x, ghost_atom_indices):
    d = jnp.abs(pred_frac_eps_x - target_frac_eps_x)
    r = d - jnp.floor(d)
    w = jnp.minimum(r, 1.0 - r)
    s_row = jnp.sum(w * w, axis=1).reshape(256, 128)
    gidx = ghost_atom_indices.astype(jnp.int32)

    out = pl.pallas_call(
        _loss_kernel,
        out_shape=jax.ShapeDtypeStruct((1, 1), jnp.float32),
        out_specs=pl.BlockSpec(memory_space=pltpu.SMEM),
    )(s_row, gidx)
    return out.reshape(())
